# baseline (speedup 1.0000x reference)
import jax
import jax.numpy as jnp
from jax import lax
from jax.experimental import pallas as pl
from jax.experimental.pallas import tpu as pltpu

N_ROWS = 20000
NUM_CLASSES = 80
N_KEEP = 1000
CONF = 0.25
BOX_D = 4
PADDED = 1024


NSPLIT = 5
CHUNK = NUM_CLASSES // NSPLIT


def _tc_body(prt_hbm, bxt_hbm, out_ref, pr_v, bx_v, bsem, *sems):
    cps = [
        pltpu.make_async_copy(
            prt_hbm.at[pl.ds(i * CHUNK, CHUNK), pl.ds(0, PADDED)],
            pr_v.at[pl.ds(i * CHUNK, CHUNK), :], sems[i])
        for i in range(NSPLIT)
    ]
    cpb = pltpu.make_async_copy(bxt_hbm.at[:, pl.ds(0, PADDED)], bx_v, bsem)
    for cp in cps:
        cp.start()
    cpb.start()

    scores = None
    for i, cp in enumerate(cps):
        cp.wait()
        m = jnp.max(pr_v[pl.ds(i * CHUNK, CHUNK), :], axis=0, keepdims=True)
        scores = m if scores is None else jnp.maximum(scores, m)
    idx = lax.broadcasted_iota(jnp.int32, (1, PADDED), 1)
    live = idx < N_KEEP
    failc = jnp.where((scores < CONF) & live, idx, N_KEEP)
    gfail = jnp.min(failc)
    keep = idx < gfail
    cpb.wait()
    bsum = jnp.sum(bx_v[...], axis=0, keepdims=True)
    total = jnp.sum(jnp.where(keep, scores + bsum, jnp.float32(0.0)))
    out_ref[0, 0] = total


@jax.jit
def kernel(post_result, pre_post_boxes):
    out = pl.pallas_call(
        _tc_body,
        out_shape=jax.ShapeDtypeStruct((1, 1), jnp.float32),
        compiler_params=pltpu.CompilerParams(vmem_limit_bytes=2 * 1024 * 1024),
        in_specs=[
            pl.BlockSpec(memory_space=pltpu.HBM),
            pl.BlockSpec(memory_space=pltpu.HBM),
        ],
        out_specs=pl.BlockSpec(memory_space=pltpu.SMEM),
        scratch_shapes=[
            pltpu.VMEM((NUM_CLASSES, PADDED), jnp.float32),
            pltpu.VMEM((BOX_D, PADDED), jnp.float32),
        ] + [pltpu.SemaphoreType.DMA] * (NSPLIT + 1),
    )(
        pltpu.with_memory_space_constraint(post_result.T, pltpu.HBM),
        pltpu.with_memory_space_constraint(pre_post_boxes.T, pltpu.HBM),
    )
    return out[0, 0]


# speedup vs baseline: 1.0310x; 1.0310x over previous
import jax
import jax.numpy as jnp
from jax import lax
from jax.experimental import pallas as pl
from jax.experimental.pallas import tpu as pltpu

N_ROWS = 20000
NUM_CLASSES = 80
N_KEEP = 1000
CONF = 0.25
BOX_D = 4
PADDED = 1024


HALF = NUM_CLASSES // 2


def _tc_body(prt_hbm, bxt_hbm, out_ref, pr_v, bx_v, sem1, sem2, sem3):
    cp1 = pltpu.make_async_copy(
        prt_hbm.at[pl.ds(0, HALF), pl.ds(0, PADDED)],
        pr_v.at[pl.ds(0, HALF), :], sem1)
    cp2 = pltpu.make_async_copy(
        prt_hbm.at[pl.ds(HALF, HALF), pl.ds(0, PADDED)],
        pr_v.at[pl.ds(HALF, HALF), :], sem2)
    cp3 = pltpu.make_async_copy(bxt_hbm.at[:, pl.ds(0, PADDED)], bx_v, sem3)
    cp1.start()
    cp2.start()
    cp3.start()

    cp1.wait()
    smax0 = jnp.max(pr_v[pl.ds(0, HALF), :], axis=0, keepdims=True)
    cp2.wait()
    smax1 = jnp.max(pr_v[pl.ds(HALF, HALF), :], axis=0, keepdims=True)
    scores = jnp.maximum(smax0, smax1)
    idx = lax.broadcasted_iota(jnp.int32, (1, PADDED), 1)
    live = idx < N_KEEP
    failc = jnp.where((scores < CONF) & live, idx, N_KEEP)
    gfail = jnp.min(failc)
    keep = idx < gfail
    cp3.wait()
    bsum = jnp.sum(bx_v[...], axis=0, keepdims=True)
    total = jnp.sum(jnp.where(keep, scores + bsum, jnp.float32(0.0)))
    out_ref[0, 0] = total


@jax.jit
def kernel(post_result, pre_post_boxes):
    out = pl.pallas_call(
        _tc_body,
        out_shape=jax.ShapeDtypeStruct((1, 1), jnp.float32),
        compiler_params=pltpu.CompilerParams(vmem_limit_bytes=2 * 1024 * 1024),
        in_specs=[
            pl.BlockSpec(memory_space=pltpu.HBM),
            pl.BlockSpec(memory_space=pltpu.HBM),
        ],
        out_specs=pl.BlockSpec(memory_space=pltpu.SMEM),
        scratch_shapes=[
            pltpu.VMEM((NUM_CLASSES, PADDED), jnp.float32),
            pltpu.VMEM((BOX_D, PADDED), jnp.float32),
            pltpu.SemaphoreType.DMA,
            pltpu.SemaphoreType.DMA,
            pltpu.SemaphoreType.DMA,
        ],
    )(
        pltpu.with_memory_space_constraint(post_result.T, pltpu.HBM),
        pltpu.with_memory_space_constraint(pre_post_boxes.T, pltpu.HBM),
    )
    return out[0, 0]
